# Initial kernel scaffold; baseline (speedup 1.0000x reference)
import jax
import jax.numpy as jnp
from jax import lax
from jax.experimental import pallas as pl
from jax.experimental.pallas import tpu as pltpu
from jax.experimental.pallas import tpu_sc as plsc

EMBED = 128
NC, NS = 2, 16
NW = NC * NS
B_TOTAL = 4096 * 50
B_PER_W = B_TOTAL // NW
CHUNK = 128
N_CHUNKS = B_PER_W // CHUNK


def _body(idx_hbm, table_hbm, out_hbm, idx_v, rows_v, sem):
    wid = lax.axis_index("s") * NC + lax.axis_index("c")
    pltpu.sync_copy(idx_hbm.at[pl.ds(wid * N_CHUNKS, N_CHUNKS)], idx_v)

    @pl.loop(0, N_CHUNKS)
    def _step(j):
        pltpu.async_copy(table_hbm.at[idx_v.at[j]], rows_v, sem).wait()
        pltpu.sync_copy(rows_v, out_hbm.at[pl.ds(wid * B_PER_W + j * CHUNK, CHUNK)])


def kernel(input, table):
    idx = input.reshape(NW * N_CHUNKS, CHUNK).astype(jnp.int32)
    mesh = plsc.VectorSubcoreMesh(
        core_axis_name="c", subcore_axis_name="s", num_cores=NC, num_subcores=NS
    )
    out = pl.kernel(
        _body,
        out_type=jax.ShapeDtypeStruct((B_TOTAL, EMBED), jnp.float32),
        mesh=mesh,
        scratch_types=[
            pltpu.VMEM((N_CHUNKS, CHUNK), jnp.int32),
            pltpu.VMEM((CHUNK, EMBED), jnp.float32),
            pltpu.SemaphoreType.DMA,
        ],
    )(idx, table)
    return out.reshape(input.shape[0], input.shape[1], EMBED)


# speedup vs baseline: 2.9798x; 2.9798x over previous
import jax
import jax.numpy as jnp
from jax import lax
from jax.experimental import pallas as pl
from jax.experimental.pallas import tpu as pltpu
from jax.experimental.pallas import tpu_sc as plsc

EMBED = 128
NC, NS = 2, 16
NW = NC * NS
B_TOTAL = 4096 * 50
B_PER_W = B_TOTAL // NW
CHUNK = 128
N_CHUNKS = B_PER_W // CHUNK


def _body(idx_hbm, table_hbm, out_hbm, idx_v, rows_v, sem):
    wid = lax.axis_index("s") * NC + lax.axis_index("c")
    pltpu.sync_copy(idx_hbm.at[wid], idx_v)

    @pl.loop(0, N_CHUNKS)
    def _step(j):
        pltpu.async_copy(table_hbm.at[idx_v.at[j]], rows_v, sem).wait()
        pltpu.sync_copy(rows_v, out_hbm.at[pl.ds(wid * B_PER_W + j * CHUNK, CHUNK)])


def kernel(input, table):
    idx = input.reshape(NW, N_CHUNKS, CHUNK).astype(jnp.int32)
    mesh = plsc.VectorSubcoreMesh(
        core_axis_name="c", subcore_axis_name="s", num_cores=NC, num_subcores=NS
    )
    out = pl.kernel(
        _body,
        out_type=jax.ShapeDtypeStruct((B_TOTAL, EMBED), jnp.float32),
        mesh=mesh,
        scratch_types=[
            pltpu.VMEM((N_CHUNKS, CHUNK), jnp.int32),
            pltpu.VMEM((CHUNK, EMBED), jnp.float32),
            pltpu.SemaphoreType.DMA,
        ],
    )(idx, table)
    return out.reshape(input.shape[0], input.shape[1], EMBED)


# speedup vs baseline: 3.3067x; 1.1097x over previous
import jax
import jax.numpy as jnp
from jax import lax
from jax.experimental import pallas as pl
from jax.experimental.pallas import tpu as pltpu
from jax.experimental.pallas import tpu_sc as plsc

EMBED = 128
NC, NS = 2, 16
NW = NC * NS
B_TOTAL = 4096 * 50
B_PER_W = B_TOTAL // NW
CHUNK = 128
N_CHUNKS = B_PER_W // CHUNK
NBUF = 5
ROUNDS = N_CHUNKS // NBUF


def _body(idx_hbm, table_hbm, out_hbm, idx_v, rows_v, gsem, ssem):
    wid = lax.axis_index("s") * NC + lax.axis_index("c")
    out_base = wid * B_PER_W
    pltpu.sync_copy(idx_hbm.at[wid], idx_v)

    for b in range(NBUF):
        pltpu.async_copy(table_hbm.at[idx_v.at[b]], rows_v.at[b], gsem[b])
    for b in range(NBUF):
        pltpu.make_async_copy(table_hbm.at[idx_v.at[b]], rows_v.at[b], gsem[b]).wait()
        pltpu.async_copy(
            rows_v.at[b], out_hbm.at[pl.ds(out_base + b * CHUNK, CHUNK)], ssem[b]
        )

    @pl.loop(1, ROUNDS)
    def _round(r):
        j0 = r * NBUF
        for b in range(NBUF):
            j = j0 + b
            pltpu.make_async_copy(
                rows_v.at[b], out_hbm.at[pl.ds(out_base + (j - NBUF) * CHUNK, CHUNK)],
                ssem[b],
            ).wait()
            pltpu.async_copy(table_hbm.at[idx_v.at[j]], rows_v.at[b], gsem[b])
        for b in range(NBUF):
            j = j0 + b
            pltpu.make_async_copy(table_hbm.at[idx_v.at[j]], rows_v.at[b], gsem[b]).wait()
            pltpu.async_copy(
                rows_v.at[b], out_hbm.at[pl.ds(out_base + j * CHUNK, CHUNK)], ssem[b]
            )

    for b in range(NBUF):
        j = (ROUNDS - 1) * NBUF + b
        pltpu.make_async_copy(
            rows_v.at[b], out_hbm.at[pl.ds(out_base + j * CHUNK, CHUNK)], ssem[b]
        ).wait()


def kernel(input, table):
    idx = input.reshape(NW, N_CHUNKS, CHUNK).astype(jnp.int32)
    mesh = plsc.VectorSubcoreMesh(
        core_axis_name="c", subcore_axis_name="s", num_cores=NC, num_subcores=NS
    )
    out = pl.kernel(
        _body,
        out_type=jax.ShapeDtypeStruct((B_TOTAL, EMBED), jnp.float32),
        mesh=mesh,
        scratch_types=[
            pltpu.VMEM((N_CHUNKS, CHUNK), jnp.int32),
            pltpu.VMEM((NBUF, CHUNK, EMBED), jnp.float32),
            [pltpu.SemaphoreType.DMA] * NBUF,
            [pltpu.SemaphoreType.DMA] * NBUF,
        ],
    )(idx, table)
    return out.reshape(input.shape[0], input.shape[1], EMBED)


# speedup vs baseline: 5.8668x; 1.7742x over previous
import jax
import jax.numpy as jnp
from jax import lax
from jax.experimental import pallas as pl
from jax.experimental.pallas import tpu as pltpu
from jax.experimental.pallas import tpu_sc as plsc

EMBED = 128
HIST = 50
BATCH = 4096
NC, NS = 2, 16
NW = NC * NS
B_PER_W = BATCH // NW
G = 4
N_CHUNKS = B_PER_W // G
NBUF = 4
ROUNDS = N_CHUNKS // NBUF


def _gather(table_hbm, idx_v, rows_v, gsem, b, c):
    for g in range(G):
        pltpu.async_copy(table_hbm.at[idx_v.at[c * G + g]], rows_v.at[b, g], gsem[b])


def _wait_gather(table_hbm, idx_v, rows_v, gsem, b, c):
    for g in range(G):
        pltpu.make_async_copy(
            table_hbm.at[idx_v.at[c * G + g]], rows_v.at[b, g], gsem[b]
        ).wait()


def _body(idx_hbm, table_hbm, out_hbm, idx_v, rows_v, gsem, ssem):
    wid = lax.axis_index("s") * NC + lax.axis_index("c")
    out_base = wid * B_PER_W
    pltpu.sync_copy(idx_hbm.at[wid], idx_v)

    for b in range(NBUF):
        _gather(table_hbm, idx_v, rows_v, gsem, b, b)
    for b in range(NBUF):
        _wait_gather(table_hbm, idx_v, rows_v, gsem, b, b)
        pltpu.async_copy(
            rows_v.at[b], out_hbm.at[pl.ds(out_base + b * G, G)], ssem[b]
        )

    @pl.loop(1, ROUNDS)
    def _round(r):
        c0 = r * NBUF
        for b in range(NBUF):
            c = c0 + b
            pltpu.make_async_copy(
                rows_v.at[b], out_hbm.at[pl.ds(out_base + (c - NBUF) * G, G)], ssem[b]
            ).wait()
            _gather(table_hbm, idx_v, rows_v, gsem, b, c)
        for b in range(NBUF):
            c = c0 + b
            _wait_gather(table_hbm, idx_v, rows_v, gsem, b, c)
            pltpu.async_copy(
                rows_v.at[b], out_hbm.at[pl.ds(out_base + c * G, G)], ssem[b]
            )

    for b in range(NBUF):
        c = (ROUNDS - 1) * NBUF + b
        pltpu.make_async_copy(
            rows_v.at[b], out_hbm.at[pl.ds(out_base + c * G, G)], ssem[b]
        ).wait()


def kernel(input, table):
    idx = input.reshape(NW, B_PER_W, HIST).astype(jnp.int32)
    mesh = plsc.VectorSubcoreMesh(
        core_axis_name="c", subcore_axis_name="s", num_cores=NC, num_subcores=NS
    )
    return pl.kernel(
        _body,
        out_type=jax.ShapeDtypeStruct((BATCH, HIST, EMBED), jnp.float32),
        mesh=mesh,
        scratch_types=[
            pltpu.VMEM((B_PER_W, HIST), jnp.int32),
            pltpu.VMEM((NBUF, G, HIST, EMBED), jnp.float32),
            [pltpu.SemaphoreType.DMA] * NBUF,
            [pltpu.SemaphoreType.DMA] * NBUF,
        ],
    )(idx, table)


# speedup vs baseline: 10.1227x; 1.7254x over previous
import jax
import jax.numpy as jnp
from jax import lax
from jax.experimental import pallas as pl
from jax.experimental.pallas import tpu as pltpu
from jax.experimental.pallas import tpu_sc as plsc

EMBED = 128
HIST = 50
BATCH = 4096
NC, NS = 2, 16
NW = NC * NS
B_TOTAL = BATCH * HIST
B_PER_W = B_TOTAL // NW
CHUNK = 128
N_CHUNKS = B_PER_W // CHUNK
NBUF = 5
ROUNDS = N_CHUNKS // NBUF


def _body(idx_hbm, table_hbm, out_hbm, idx_v, rows_v, gsem, ssem):
    wid = lax.axis_index("s") * NC + lax.axis_index("c")
    out_base = wid * B_PER_W
    pltpu.sync_copy(idx_hbm.at[wid], idx_v)

    for b in range(NBUF):
        pltpu.async_copy(table_hbm.at[idx_v.at[b]], rows_v.at[b], gsem[b])
    for b in range(NBUF):
        pltpu.make_async_copy(table_hbm.at[idx_v.at[b]], rows_v.at[b], gsem[b]).wait()
        pltpu.async_copy(
            rows_v.at[b], out_hbm.at[pl.ds(out_base + b * CHUNK, CHUNK)], ssem[b]
        )

    @pl.loop(1, ROUNDS)
    def _round(r):
        j0 = r * NBUF
        for b in range(NBUF):
            j = j0 + b
            pltpu.make_async_copy(
                rows_v.at[b], out_hbm.at[pl.ds(out_base + (j - NBUF) * CHUNK, CHUNK)],
                ssem[b],
            ).wait()
            pltpu.async_copy(table_hbm.at[idx_v.at[j]], rows_v.at[b], gsem[b])
        for b in range(NBUF):
            j = j0 + b
            pltpu.make_async_copy(table_hbm.at[idx_v.at[j]], rows_v.at[b], gsem[b]).wait()
            pltpu.async_copy(
                rows_v.at[b], out_hbm.at[pl.ds(out_base + j * CHUNK, CHUNK)], ssem[b]
            )

    for b in range(NBUF):
        j = (ROUNDS - 1) * NBUF + b
        pltpu.make_async_copy(
            rows_v.at[b], out_hbm.at[pl.ds(out_base + j * CHUNK, CHUNK)], ssem[b]
        ).wait()


def kernel(input, table):
    idx = input.T.reshape(NW, N_CHUNKS, CHUNK).astype(jnp.int32)
    mesh = plsc.VectorSubcoreMesh(
        core_axis_name="c", subcore_axis_name="s", num_cores=NC, num_subcores=NS
    )
    flat = pl.kernel(
        _body,
        out_type=jax.ShapeDtypeStruct((B_TOTAL, EMBED), jnp.float32),
        mesh=mesh,
        scratch_types=[
            pltpu.VMEM((N_CHUNKS, CHUNK), jnp.int32),
            pltpu.VMEM((NBUF, CHUNK, EMBED), jnp.float32),
            [pltpu.SemaphoreType.DMA] * NBUF,
            [pltpu.SemaphoreType.DMA] * NBUF,
        ],
    )(idx, table)
    return flat.reshape(HIST, BATCH, EMBED).transpose(1, 0, 2)


# speedup vs baseline: 10.4205x; 1.0294x over previous
import jax
import jax.numpy as jnp
from jax import lax
from jax.experimental import pallas as pl
from jax.experimental.pallas import tpu as pltpu
from jax.experimental.pallas import tpu_sc as plsc

EMBED = 128
HIST = 50
BATCH = 4096
NC, NS = 2, 16
NW = NC * NS
B_TOTAL = BATCH * HIST
CHUNK = 128
N_CHUNKS = HIST
NBUF = 5
ROUNDS = N_CHUNKS // NBUF


def _body(idx_hbm, table_hbm, out_hbm, idx_v, rows_v, gsem, ssem):
    wid = lax.axis_index("s") * NC + lax.axis_index("c")
    col = wid * CHUNK
    pltpu.sync_copy(idx_hbm.at[:, pl.ds(col, CHUNK)], idx_v)

    for b in range(NBUF):
        pltpu.async_copy(table_hbm.at[idx_v.at[b]], rows_v.at[b], gsem[b])
    for b in range(NBUF):
        pltpu.make_async_copy(table_hbm.at[idx_v.at[b]], rows_v.at[b], gsem[b]).wait()
        pltpu.async_copy(
            rows_v.at[b], out_hbm.at[pl.ds(b * BATCH + col, CHUNK)], ssem[b]
        )

    @pl.loop(1, ROUNDS)
    def _round(r):
        j0 = r * NBUF
        for b in range(NBUF):
            j = j0 + b
            pltpu.make_async_copy(
                rows_v.at[b], out_hbm.at[pl.ds((j - NBUF) * BATCH + col, CHUNK)],
                ssem[b],
            ).wait()
            pltpu.async_copy(table_hbm.at[idx_v.at[j]], rows_v.at[b], gsem[b])
        for b in range(NBUF):
            j = j0 + b
            pltpu.make_async_copy(table_hbm.at[idx_v.at[j]], rows_v.at[b], gsem[b]).wait()
            pltpu.async_copy(
                rows_v.at[b], out_hbm.at[pl.ds(j * BATCH + col, CHUNK)], ssem[b]
            )

    for b in range(NBUF):
        j = (ROUNDS - 1) * NBUF + b
        pltpu.make_async_copy(
            rows_v.at[b], out_hbm.at[pl.ds(j * BATCH + col, CHUNK)], ssem[b]
        ).wait()


def kernel(input, table):
    idx = input.T.astype(jnp.int32)
    mesh = plsc.VectorSubcoreMesh(
        core_axis_name="c", subcore_axis_name="s", num_cores=NC, num_subcores=NS
    )
    flat = pl.kernel(
        _body,
        out_type=jax.ShapeDtypeStruct((B_TOTAL, EMBED), jnp.float32),
        mesh=mesh,
        scratch_types=[
            pltpu.VMEM((N_CHUNKS, CHUNK), jnp.int32),
            pltpu.VMEM((NBUF, CHUNK, EMBED), jnp.float32),
            [pltpu.SemaphoreType.DMA] * NBUF,
            [pltpu.SemaphoreType.DMA] * NBUF,
        ],
    )(idx, table)
    return flat.reshape(HIST, BATCH, EMBED).transpose(1, 0, 2)
